# baseline (speedup 1.0000x reference)
import functools

import jax
import jax.numpy as jnp
from jax import lax
from jax.experimental import pallas as pl
from jax.experimental.pallas import tpu as pltpu
from jax.experimental.pallas import tpu_sc as plsc

VOCAB = 1000000
D = 64
B = 4096
L = 200
N = B * L

NC = 2
NS = 16
NW = NC * NS
PER_W = N // NW
NBUF = 2
CHUNK = 400
GRP = CHUNK // 16
N_CHUNKS = PER_W // CHUNK
N_ITERS = N_CHUNKS // NBUF


def _emb_body(idx_hbm, lbl_hbm, emb_hbm, seg_hbm, out_hbm,
              idx_v, lbl_v, tok_v, seg_v, sem_g, sem_s, sem_w):
    wid = lax.axis_index("s") * NC + lax.axis_index("c")
    base = wid * PER_W

    def stage(g, b):
        start = base + g * CHUNK
        pltpu.sync_copy(idx_hbm.at[pl.ds(start, CHUNK)], idx_v.at[b])
        pltpu.sync_copy(lbl_hbm.at[pl.ds(start, CHUNK)], lbl_v.at[b])

        def fire(p, c):
            sl = pl.ds(p * 16, 16)
            pltpu.async_copy(seg_hbm.at[lbl_v[b, sl]], seg_v.at[b, sl],
                             sem_s.at[b])
            pltpu.async_copy(emb_hbm.at[idx_v[b, sl]], tok_v.at[b, sl],
                             sem_g.at[b])
            return c

        lax.fori_loop(0, GRP, fire, 0)

    def finish(g, b):
        pltpu.make_async_copy(seg_hbm.at[lbl_v.at[b]], seg_v.at[b],
                              sem_s.at[b]).wait()
        pltpu.make_async_copy(emb_hbm.at[idx_v.at[b]], tok_v.at[b],
                              sem_g.at[b]).wait()

        def add_step(p, c):
            for k in range(D // 16):
                sl = pl.ds(k * 16, 16)
                tok_v[b, p, sl] = tok_v[b, p, sl] + seg_v[b, p, sl]
            return c

        lax.fori_loop(0, CHUNK, add_step, 0, unroll=2)
        start = base + g * CHUNK
        pltpu.async_copy(tok_v.at[b], out_hbm.at[pl.ds(start, CHUNK)],
                         sem_w.at[b]).wait()

    stage(0, 0)

    def step(i, carry):
        g0 = i * NBUF
        stage(g0 + 1, 1)
        finish(g0, 0)

        @pl.when(i + 1 < N_ITERS)
        def _():
            stage(g0 + 2, 0)

        finish(g0 + 1, 1)
        return carry

    lax.fori_loop(0, N_ITERS, step, 0)


@jax.jit
def _emb_lookup(idx, lbl, emb_table, seg_table):
    mesh = plsc.VectorSubcoreMesh(core_axis_name="c", subcore_axis_name="s")
    f = pl.kernel(
        _emb_body,
        out_type=jax.ShapeDtypeStruct((N, D), jnp.float32),
        mesh=mesh,
        scratch_types=[
            pltpu.VMEM((NBUF, CHUNK), jnp.int32),
            pltpu.VMEM((NBUF, CHUNK), jnp.int32),
            pltpu.VMEM((NBUF, CHUNK, D), jnp.float32),
            pltpu.VMEM((NBUF, CHUNK, D), jnp.float32),
            pltpu.SemaphoreType.DMA((NBUF,)),
            pltpu.SemaphoreType.DMA((NBUF,)),
            pltpu.SemaphoreType.DMA((NBUF,)),
        ],
        compiler_params=pltpu.CompilerParams(use_tc_tiling_on_sc=False),
    )
    return f(idx, lbl, emb_table, seg_table)


def kernel(x, segment_label, emb_table, seg_table):
    idx = x.reshape(-1).astype(jnp.int32)
    lbl = segment_label.reshape(-1).astype(jnp.int32)
    out = _emb_lookup(idx, lbl, emb_table, seg_table)
    return out.reshape(B, L, D)


# speedup vs baseline: 1.0026x; 1.0026x over previous
import functools

import jax
import jax.numpy as jnp
from jax import lax
from jax.experimental import pallas as pl
from jax.experimental.pallas import tpu as pltpu
from jax.experimental.pallas import tpu_sc as plsc

VOCAB = 1000000
D = 64
B = 4096
L = 200
N = B * L

NC = 2
NS = 16
NW = NC * NS
PER_W = N // NW
NBUF = 2
CHUNK = 400
GRP = CHUNK // 16
N_CHUNKS = PER_W // CHUNK
N_ITERS = N_CHUNKS // NBUF


def _emb_body(idx_hbm, lbl_hbm, emb_hbm, seg_hbm, out_hbm,
              idx_v, lbl_v, tok_v, seg_v, sem_g, sem_s, sem_w):
    wid = lax.axis_index("s") * NC + lax.axis_index("c")
    base = wid * PER_W

    def stage(g, b):
        start = base + g * CHUNK
        with jax.named_scope("stg_idx"):
            pltpu.sync_copy(idx_hbm.at[pl.ds(start, CHUNK)], idx_v.at[b])
            pltpu.sync_copy(lbl_hbm.at[pl.ds(start, CHUNK)], lbl_v.at[b])

        def fire(p, c):
            sl = pl.ds(p * 16, 16)
            pltpu.async_copy(seg_hbm.at[lbl_v[b, sl]], seg_v.at[b, sl],
                             sem_s.at[b])
            pltpu.async_copy(emb_hbm.at[idx_v[b, sl]], tok_v.at[b, sl],
                             sem_g.at[b])
            return c

        with jax.named_scope("fire"):
            lax.fori_loop(0, GRP, fire, 0)

    def finish(g, b):
        with jax.named_scope("drain"):
            pltpu.make_async_copy(seg_hbm.at[lbl_v.at[b]], seg_v.at[b],
                                  sem_s.at[b]).wait()
            pltpu.make_async_copy(emb_hbm.at[idx_v.at[b]], tok_v.at[b],
                                  sem_g.at[b]).wait()

        def add_step(p, c):
            for k in range(D // 16):
                sl = pl.ds(k * 16, 16)
                tok_v[b, p, sl] = tok_v[b, p, sl] + seg_v[b, p, sl]
            return c

        with jax.named_scope("addloop"):
            lax.fori_loop(0, CHUNK, add_step, 0, unroll=2)
        start = base + g * CHUNK
        with jax.named_scope("wb"):
            pltpu.async_copy(tok_v.at[b], out_hbm.at[pl.ds(start, CHUNK)],
                             sem_w.at[b]).wait()

    stage(0, 0)

    def step(i, carry):
        g0 = i * NBUF
        stage(g0 + 1, 1)
        finish(g0, 0)

        @pl.when(i + 1 < N_ITERS)
        def _():
            stage(g0 + 2, 0)

        finish(g0 + 1, 1)
        return carry

    lax.fori_loop(0, N_ITERS, step, 0)


@jax.jit
def _emb_lookup(idx, lbl, emb_table, seg_table):
    mesh = plsc.VectorSubcoreMesh(core_axis_name="c", subcore_axis_name="s")
    f = pl.kernel(
        _emb_body,
        out_type=jax.ShapeDtypeStruct((N, D), jnp.float32),
        mesh=mesh,
        scratch_types=[
            pltpu.VMEM((NBUF, CHUNK), jnp.int32),
            pltpu.VMEM((NBUF, CHUNK), jnp.int32),
            pltpu.VMEM((NBUF, CHUNK, D), jnp.float32),
            pltpu.VMEM((NBUF, CHUNK, D), jnp.float32),
            pltpu.SemaphoreType.DMA((NBUF,)),
            pltpu.SemaphoreType.DMA((NBUF,)),
            pltpu.SemaphoreType.DMA((NBUF,)),
        ],
        compiler_params=pltpu.CompilerParams(use_tc_tiling_on_sc=False),
    )
    return f(idx, lbl, emb_table, seg_table)


def kernel(x, segment_label, emb_table, seg_table):
    idx = x.reshape(-1).astype(jnp.int32)
    lbl = segment_label.reshape(-1).astype(jnp.int32)
    out = _emb_lookup(idx, lbl, emb_table, seg_table)
    return out.reshape(B, L, D)
